# baseline (speedup 1.0000x reference)
import functools

import jax
import jax.numpy as jnp
from jax import lax
from jax.experimental import pallas as pl
from jax.experimental.pallas import tpu as pltpu
from jax.experimental.pallas import tpu_sc as plsc

ROWS = 64 * 512
VEC_PAD = 128
VEC_I32 = VEC_PAD // 2
D_MODEL = 1024

NC = 2
NS = 16
NW = NC * NS
RPW = ROWS // NW
CH = 128
NCH = RPW // CH

_sc_mesh = plsc.VectorSubcoreMesh(core_axis_name="c", subcore_axis_name="s")


@functools.partial(
    pl.kernel,
    mesh=_sc_mesh,
    out_type=jax.ShapeDtypeStruct((ROWS, VEC_I32), jnp.int32),
    scratch_types=[
        pltpu.VMEM((NCH, CH), jnp.int32),
        pltpu.VMEM((CH, VEC_I32), jnp.int32),
        pltpu.VMEM((CH, VEC_I32), jnp.int32),
        pltpu.SemaphoreType.DMA,
        pltpu.SemaphoreType.DMA,
    ],
    compiler_params=pltpu.CompilerParams(use_tc_tiling_on_sc=False),
)
def _sc_gather(table_hbm, idx_hbm, out_hbm, idx_v, buf0, buf1, sem0, sem1):
    wid = lax.axis_index("s") * NC + lax.axis_index("c")
    base = wid * RPW
    pltpu.sync_copy(idx_hbm.at[wid], idx_v)
    bufs = (buf0, buf1)
    sems = (sem0, sem1)
    handles = [None, None]
    handles[0] = pltpu.async_copy(table_hbm.at[idx_v.at[0]], buf0, sem0)
    for j in range(NCH):
        cur = j % 2
        if j + 1 < NCH:
            nxt = (j + 1) % 2
            handles[nxt] = pltpu.async_copy(
                table_hbm.at[idx_v.at[j + 1]], bufs[nxt], sems[nxt])
        handles[cur].wait()
        pltpu.sync_copy(bufs[cur], out_hbm.at[pl.ds(base + j * CH, CH)])


_MM_BM = 2048


def _mm_body(x_ref, w_ref, b_ref, o_ref):
    o_ref[...] = (
        jnp.dot(x_ref[...], w_ref[...], preferred_element_type=jnp.float32)
        + b_ref[...]
    )


@jax.jit
def _tc_matmul(x, w, bvec):
    return pl.pallas_call(
        _mm_body,
        grid=(ROWS // _MM_BM,),
        in_specs=[
            pl.BlockSpec((_MM_BM, VEC_PAD), lambda i: (i, 0)),
            pl.BlockSpec((VEC_PAD, D_MODEL), lambda i: (0, 0)),
            pl.BlockSpec((1, D_MODEL), lambda i: (0, 0)),
        ],
        out_specs=pl.BlockSpec((_MM_BM, D_MODEL), lambda i: (i, 0)),
        out_shape=jax.ShapeDtypeStruct((ROWS, D_MODEL), jnp.float32),
    )(x, w, bvec)


def kernel(protX, table, W, b):
    B, L = protX.shape
    vocab, vec = table.shape
    d_model = W.shape[1]
    idx = protX.reshape(NW, NCH, CH).astype(jnp.int32)
    table_pad = jnp.pad(table.astype(jnp.bfloat16), ((0, 0), (0, VEC_PAD - vec)))
    table_i32 = jax.lax.bitcast_convert_type(
        table_pad.reshape(vocab, VEC_I32, 2), jnp.int32)
    w_pad = jnp.pad(W.astype(jnp.bfloat16), ((0, VEC_PAD - vec), (0, 0)))
    gathered_i32 = _sc_gather(table_i32, idx)
    gathered = jax.lax.bitcast_convert_type(
        gathered_i32, jnp.bfloat16).reshape(ROWS, VEC_PAD)
    emb = _tc_matmul(gathered, w_pad, b.reshape(1, d_model))
    return emb.reshape(B, L, d_model)


# speedup vs baseline: 2.1740x; 2.1740x over previous
import functools

import jax
import jax.numpy as jnp
from jax import lax
from jax.experimental import pallas as pl
from jax.experimental.pallas import tpu as pltpu
from jax.experimental.pallas import tpu_sc as plsc

ROWS = 64 * 512
VEC_PAD = 128
D_MODEL = 1024

NC = 2
NS = 16
NW = NC * NS
RPW = ROWS // NW
CH = 128
NCH = RPW // CH
NBUF = 4

_sc_mesh = plsc.VectorSubcoreMesh(core_axis_name="c", subcore_axis_name="s")


@functools.partial(
    pl.kernel,
    mesh=_sc_mesh,
    out_type=jax.ShapeDtypeStruct((ROWS, VEC_PAD), jnp.float32),
    scratch_types=[
        pltpu.VMEM((NCH, CH), jnp.int32),
        [pltpu.VMEM((CH, VEC_PAD), jnp.float32) for _ in range(NBUF)],
        [pltpu.SemaphoreType.DMA for _ in range(NBUF)],
        [pltpu.SemaphoreType.DMA for _ in range(NBUF)],
    ],
)
def _sc_gather(table_hbm, idx_hbm, out_hbm, idx_v, bufs, gsems, ssems):
    wid = lax.axis_index("s") * NC + lax.axis_index("c")
    base = wid * RPW
    pltpu.sync_copy(idx_hbm.at[wid], idx_v)

    def gather(j):
        return pltpu.async_copy(
            table_hbm.at[idx_v.at[j]], bufs[j % NBUF], gsems[j % NBUF])

    def scatter(j):
        return pltpu.async_copy(
            bufs[j % NBUF], out_hbm.at[pl.ds(base + j * CH, CH)],
            ssems[j % NBUF])

    gh = [None] * NCH
    sh = [None] * NCH
    drained = set()
    for j in range(min(NBUF - 1, NCH)):
        gh[j] = gather(j)
    for j in range(NCH):
        gh[j].wait()
        sh[j] = scatter(j)
        nxt = j + NBUF - 1
        if nxt < NCH:
            if j >= 1:
                sh[j - 1].wait()
                drained.add(j - 1)
            gh[nxt] = gather(nxt)
    for j in range(NCH):
        if j not in drained:
            sh[j].wait()


_MM_BM = 2048


def _mm_body(x_ref, w_ref, b_ref, o_ref):
    o_ref[...] = (
        jnp.dot(x_ref[...], w_ref[...], preferred_element_type=jnp.float32)
        + b_ref[...]
    )


@jax.jit
def _tc_matmul(x, w, bvec):
    return pl.pallas_call(
        _mm_body,
        grid=(ROWS // _MM_BM,),
        in_specs=[
            pl.BlockSpec((_MM_BM, VEC_PAD), lambda i: (i, 0)),
            pl.BlockSpec((VEC_PAD, D_MODEL), lambda i: (0, 0)),
            pl.BlockSpec((1, D_MODEL), lambda i: (0, 0)),
        ],
        out_specs=pl.BlockSpec((_MM_BM, D_MODEL), lambda i: (i, 0)),
        out_shape=jax.ShapeDtypeStruct((ROWS, D_MODEL), jnp.float32),
    )(x, w, bvec)


def kernel(protX, table, W, b):
    B, L = protX.shape
    vocab, vec = table.shape
    d_model = W.shape[1]
    idx = protX.reshape(NW, NCH, CH).astype(jnp.int32)
    table_pad = jnp.pad(table, ((0, 0), (0, VEC_PAD - vec)))
    w_pad = jnp.pad(W, ((0, VEC_PAD - vec), (0, 0)))
    gathered = _sc_gather(table_pad, idx)
    emb = _tc_matmul(gathered, w_pad, b.reshape(1, d_model))
    return emb.reshape(B, L, d_model)


# speedup vs baseline: 2.1938x; 1.0091x over previous
import functools

import jax
import jax.numpy as jnp
from jax import lax
from jax.experimental import pallas as pl
from jax.experimental.pallas import tpu as pltpu
from jax.experimental.pallas import tpu_sc as plsc

ROWS = 64 * 512
VEC_PAD = 128
D_MODEL = 1024

NC = 2
NS = 16
NW = NC * NS
RPW = ROWS // NW
CH = 128
NCH = RPW // CH
NBUF = 7

_sc_mesh = plsc.VectorSubcoreMesh(core_axis_name="c", subcore_axis_name="s")


@functools.partial(
    pl.kernel,
    mesh=_sc_mesh,
    out_type=jax.ShapeDtypeStruct((ROWS, VEC_PAD), jnp.float32),
    scratch_types=[
        pltpu.VMEM((NCH, CH), jnp.int32),
        [pltpu.VMEM((CH, VEC_PAD), jnp.float32) for _ in range(NBUF)],
        [pltpu.SemaphoreType.DMA for _ in range(NBUF)],
        [pltpu.SemaphoreType.DMA for _ in range(NBUF)],
    ],
)
def _sc_gather(table_hbm, idx_hbm, out_hbm, idx_v, bufs, gsems, ssems):
    wid = lax.axis_index("s") * NC + lax.axis_index("c")
    base = wid * RPW
    pltpu.sync_copy(idx_hbm.at[wid], idx_v)

    def gather(j):
        return pltpu.async_copy(
            table_hbm.at[idx_v.at[j]], bufs[j % NBUF], gsems[j % NBUF])

    def scatter(j):
        return pltpu.async_copy(
            bufs[j % NBUF], out_hbm.at[pl.ds(base + j * CH, CH)],
            ssems[j % NBUF])

    gh = [None] * NCH
    sh = [None] * NCH
    drained = set()
    for j in range(min(NBUF - 1, NCH)):
        gh[j] = gather(j)
    for j in range(NCH):
        gh[j].wait()
        sh[j] = scatter(j)
        nxt = j + NBUF - 1
        if nxt < NCH:
            if j >= 1:
                sh[j - 1].wait()
                drained.add(j - 1)
            gh[nxt] = gather(nxt)
    for j in range(NCH):
        if j not in drained:
            sh[j].wait()


_MM_BM = 2048


def _mm_body(x_ref, w_ref, b_ref, o_ref):
    o_ref[...] = (
        jnp.dot(x_ref[...], w_ref[...], preferred_element_type=jnp.float32)
        + b_ref[...]
    )


@jax.jit
def _tc_matmul(x, w, bvec):
    return pl.pallas_call(
        _mm_body,
        grid=(ROWS // _MM_BM,),
        in_specs=[
            pl.BlockSpec((_MM_BM, VEC_PAD), lambda i: (i, 0)),
            pl.BlockSpec((VEC_PAD, D_MODEL), lambda i: (0, 0)),
            pl.BlockSpec((1, D_MODEL), lambda i: (0, 0)),
        ],
        out_specs=pl.BlockSpec((_MM_BM, D_MODEL), lambda i: (i, 0)),
        out_shape=jax.ShapeDtypeStruct((ROWS, D_MODEL), jnp.float32),
    )(x, w, bvec)


def kernel(protX, table, W, b):
    B, L = protX.shape
    vocab, vec = table.shape
    d_model = W.shape[1]
    idx = protX.reshape(NW, NCH, CH).astype(jnp.int32)
    table_pad = jnp.pad(table, ((0, 0), (0, VEC_PAD - vec)))
    w_pad = jnp.pad(W, ((0, VEC_PAD - vec), (0, 0)))
    gathered = _sc_gather(table_pad, idx)
    emb = _tc_matmul(gathered, w_pad, b.reshape(1, d_model))
    return emb.reshape(B, L, d_model)
